# baseline (speedup 1.0000x reference)
import functools

import jax
import jax.numpy as jnp
from jax import lax
from jax.experimental import pallas as pl
from jax.experimental.pallas import tpu as pltpu
from jax.experimental.pallas import tpu_sc as plsc

NC, NS, LANES = 2, 16, 16


def _make_kernel(BL, V, E, L, per_w, chunk):
    n_chunks = per_w // chunk
    n_periods = chunk // L
    mesh = plsc.VectorSubcoreMesh(core_axis_name="c", subcore_axis_name="s")

    @functools.partial(
        pl.kernel,
        out_type=jax.ShapeDtypeStruct((BL, E), jnp.float32),
        mesh=mesh,
        scratch_types=[
            pltpu.VMEM((chunk,), jnp.int32),
            pltpu.VMEM((chunk, E), jnp.float32),
            pltpu.VMEM((L, E), jnp.float32),
            pltpu.SemaphoreType.DMA,
        ],
        compiler_params=pltpu.CompilerParams(use_tc_tiling_on_sc=False),
    )
    def k(seq_hbm, table_hbm, pe_hbm, out_hbm, idx_v, rows_v, pe_v, sem):
        wid = lax.axis_index("s") * NC + lax.axis_index("c")
        w_base = wid * per_w

        pltpu.sync_copy(pe_hbm.at[pl.ds(0, L)], pe_v)

        def chunk_body(g, carry):
            base = w_base + g * chunk
            pltpu.sync_copy(seq_hbm.at[pl.ds(base, chunk)], idx_v)
            pltpu.async_copy(table_hbm.at[idx_v], rows_v, sem).wait()

            def period_body(p, carry2):
                def row_body(i, carry3):
                    r = p * L + i
                    for j in range(E // LANES):
                        s = pl.ds(j * LANES, LANES)
                        rows_v[r, s] = rows_v[r, s] + pe_v[i, s]
                    return carry3

                return lax.fori_loop(0, L, row_body, carry2)

            lax.fori_loop(0, n_periods, period_body, 0)
            pltpu.sync_copy(rows_v, out_hbm.at[pl.ds(base, chunk)])
            return carry

        lax.fori_loop(0, n_chunks, chunk_body, 0)

    return k


def kernel(sequence, token_table, pe):
    B, L = sequence.shape
    V, E = token_table.shape
    BL = B * L
    n_workers = NC * NS
    per_w = BL // n_workers
    chunk = 4 * L
    assert BL % n_workers == 0 and per_w % chunk == 0 and E % LANES == 0

    seq_flat = sequence.reshape(BL).astype(jnp.int32)
    out_flat = _make_kernel(BL, V, E, L, per_w, chunk)(
        seq_flat, token_table, pe
    )
    return out_flat.reshape(B, L, E)


# speedup vs baseline: 1.0863x; 1.0863x over previous
import functools

import jax
import jax.numpy as jnp
from jax import lax
from jax.experimental import pallas as pl
from jax.experimental.pallas import tpu as pltpu
from jax.experimental.pallas import tpu_sc as plsc

NC, NS, LANES = 2, 16, 16
UNROLL = 8


def _make_kernel(BL, V, E, L, per_w, chunk):
    n_chunks = per_w // chunk
    mesh = plsc.VectorSubcoreMesh(core_axis_name="c", subcore_axis_name="s")

    @functools.partial(
        pl.kernel,
        out_type=jax.ShapeDtypeStruct((BL, E), jnp.float32),
        mesh=mesh,
        scratch_types=[
            pltpu.VMEM((2, chunk), jnp.int32),
            pltpu.VMEM((2, chunk, E), jnp.float32),
            pltpu.VMEM((chunk, E), jnp.float32),
            pltpu.SemaphoreType.DMA((2,)),
            pltpu.SemaphoreType.DMA((2,)),
            pltpu.SemaphoreType.DMA((2,)),
        ],
        compiler_params=pltpu.CompilerParams(use_tc_tiling_on_sc=False),
    )
    def k(seq_hbm, table_hbm, pe_hbm, out_hbm, idx2, rows2, pet, si, sg, ss):
        wid = lax.axis_index("s") * NC + lax.axis_index("c")
        w_base = wid * per_w

        for p in range(chunk // L):
            pltpu.sync_copy(pe_hbm.at[pl.ds(0, L)], pet.at[pl.ds(p * L, L)])

        def start_idx(g, b):
            pltpu.async_copy(
                seq_hbm.at[pl.ds(w_base + g * chunk, chunk)], idx2.at[b],
                si.at[b])

        def wait_idx(b):
            pltpu.make_async_copy(
                seq_hbm.at[pl.ds(0, chunk)], idx2.at[b], si.at[b]).wait()

        def start_gather(b):
            pltpu.async_copy(table_hbm.at[idx2.at[b]], rows2.at[b], sg.at[b])

        def wait_gather(b):
            pltpu.make_async_copy(
                table_hbm.at[idx2.at[b]], rows2.at[b], sg.at[b]).wait()

        def start_store(g, b):
            pltpu.async_copy(
                rows2.at[b], out_hbm.at[pl.ds(w_base + g * chunk, chunk)],
                ss.at[b])

        def wait_store(b):
            pltpu.make_async_copy(
                rows2.at[b], out_hbm.at[pl.ds(0, chunk)], ss.at[b]).wait()

        def add_pe(b):
            def body(i, carry):
                r0 = i * UNROLL
                for u in range(UNROLL):
                    r = r0 + u
                    for j in range(E // LANES):
                        s = pl.ds(j * LANES, LANES)
                        plsc.addupdate(rows2.at[b, r, s], pet[r, s])
                return carry

            lax.fori_loop(0, chunk // UNROLL, body, 0)

        start_idx(0, 0)
        wait_idx(0)
        start_gather(0)
        start_idx(1, 1)

        def half(g, b):
            wait_idx(b)
            start_gather(b)
            wait_gather(1 - b)

            @pl.when(g + 1 < n_chunks)
            def _():
                start_idx(g + 1, 1 - b)

            add_pe(1 - b)
            start_store(g - 1, 1 - b)

        half(1, 1)

        def pair_body(p, carry):
            g = 2 * p
            wait_store(0)
            half(g, 0)
            wait_store(1)
            half(g + 1, 1)
            return carry

        lax.fori_loop(1, n_chunks // 2, pair_body, 0)

        b_last = (n_chunks - 1) % 2
        wait_gather(b_last)
        add_pe(b_last)
        start_store(n_chunks - 1, b_last)
        wait_store(1 - b_last)
        wait_store(b_last)

    return k


def kernel(sequence, token_table, pe):
    B, L = sequence.shape
    V, E = token_table.shape
    BL = B * L
    n_workers = NC * NS
    per_w = BL // n_workers
    chunk = 2 * L
    assert BL % n_workers == 0 and per_w % (2 * chunk) == 0
    assert E % LANES == 0 and chunk % UNROLL == 0

    seq_flat = sequence.reshape(BL).astype(jnp.int32)
    out_flat = _make_kernel(BL, V, E, L, per_w, chunk)(
        seq_flat, token_table, pe
    )
    return out_flat.reshape(B, L, E)


# speedup vs baseline: 1.6461x; 1.5153x over previous
import functools

import jax
import jax.numpy as jnp
from jax import lax
from jax.experimental import pallas as pl
from jax.experimental.pallas import tpu as pltpu
from jax.experimental.pallas import tpu_sc as plsc

NC, NS = 2, 16
CB = 2048
BB = 128


def _t1_table_to_rowmajor(V, E):
    nblk = (V + CB - 1) // CB

    def body(x_ref, o_ref):
        xt = x_ref[...].T.reshape(CB // 2, 2, E)
        o_ref[...] = jnp.concatenate([xt[:, 0, :], xt[:, 1, :]], axis=-1)

    return pl.pallas_call(
        body,
        grid=(nblk,),
        in_specs=[pl.BlockSpec((E, CB), lambda i: (0, i))],
        out_specs=pl.BlockSpec((CB // 2, 2 * E), lambda i: (i, 0)),
        out_shape=jax.ShapeDtypeStruct((V // 2, 2 * E), jnp.float32),
    )


def _sc_gather(BL, V, E, per_w, chunk):
    n_chunks = per_w // chunk
    mesh = plsc.VectorSubcoreMesh(core_axis_name="c", subcore_axis_name="s")

    @functools.partial(
        pl.kernel,
        out_type=jax.ShapeDtypeStruct((BL, E), jnp.float32),
        mesh=mesh,
        scratch_types=[
            pltpu.VMEM((2, chunk), jnp.int32),
            pltpu.VMEM((2, chunk, E), jnp.float32),
            pltpu.SemaphoreType.DMA((2,)),
            pltpu.SemaphoreType.DMA((2,)),
            pltpu.SemaphoreType.DMA((2,)),
        ],
        compiler_params=pltpu.CompilerParams(use_tc_tiling_on_sc=False),
    )
    def k(seq_hbm, table_hbm, out_hbm, idx2, rows2, si, sg, ss):
        wid = lax.axis_index("s") * NC + lax.axis_index("c")
        w_base = wid * per_w

        def start_idx(g, b):
            pltpu.async_copy(
                seq_hbm.at[pl.ds(w_base + g * chunk, chunk)], idx2.at[b],
                si.at[b])

        def wait_idx(b):
            pltpu.make_async_copy(
                seq_hbm.at[pl.ds(0, chunk)], idx2.at[b], si.at[b]).wait()

        def start_gather(b):
            pltpu.async_copy(table_hbm.at[idx2.at[b]], rows2.at[b], sg.at[b])

        def wait_gather(b):
            pltpu.make_async_copy(
                table_hbm.at[idx2.at[b]], rows2.at[b], sg.at[b]).wait()

        def start_store(g, b):
            pltpu.async_copy(
                rows2.at[b], out_hbm.at[pl.ds(w_base + g * chunk, chunk)],
                ss.at[b])

        def wait_store(b):
            pltpu.make_async_copy(
                rows2.at[b], out_hbm.at[pl.ds(0, chunk)], ss.at[b]).wait()

        start_idx(0, 0)
        wait_idx(0)
        start_gather(0)
        start_idx(1, 1)

        def half(g, b):
            wait_idx(b)
            start_gather(b)
            wait_gather(1 - b)

            @pl.when(g + 1 < n_chunks)
            def _():
                start_idx(g + 1, 1 - b)

            start_store(g - 1, 1 - b)

        half(1, 1)

        def pair_body(p, carry):
            g = 2 * p
            wait_store(0)
            half(g, 0)
            wait_store(1)
            half(g + 1, 1)
            return carry

        lax.fori_loop(1, n_chunks // 2, pair_body, 0)

        b_last = (n_chunks - 1) % 2
        wait_gather(b_last)
        start_store(n_chunks - 1, b_last)
        wait_store(1 - b_last)
        wait_store(b_last)

    return k


def _t2_to_native_plus_pe(B, L, E):
    P = L // 2

    def body(x_ref, pe_ref, o_ref):
        x3 = x_ref[...].reshape(BB, P, 2 * E)
        pe_v = pe_ref[...]
        for p in range(P):
            xp = x3[:, p, :].T
            o_ref[2 * p] = xp[:E, :] + pe_v[2 * p][:, None]
            o_ref[2 * p + 1] = xp[E:, :] + pe_v[2 * p + 1][:, None]

    return pl.pallas_call(
        body,
        grid=(B // BB,),
        in_specs=[
            pl.BlockSpec((BB * P, 2 * E), lambda bi: (bi, 0)),
            pl.BlockSpec((L, E), lambda bi: (0, 0)),
        ],
        out_specs=pl.BlockSpec((L, E, BB), lambda bi: (0, 0, bi)),
        out_shape=jax.ShapeDtypeStruct((L, E, B), jnp.float32),
    )


def kernel(sequence, token_table, pe):
    B, L = sequence.shape
    V, E = token_table.shape
    BL = B * L
    n_workers = NC * NS
    per_w = BL // n_workers
    chunk = 800
    assert BL % n_workers == 0 and per_w % (2 * chunk) == 0
    assert B % BB == 0 and L % 2 == 0

    table_rm = _t1_table_to_rowmajor(V, E)(token_table.T)
    table_lin = table_rm.reshape(V, E)

    seq_flat = sequence.reshape(BL).astype(jnp.int32)
    flat = _sc_gather(BL, V, E, per_w, chunk)(seq_flat, table_lin)

    t2in = flat.reshape(B * L // 2, 2 * E)
    out_T = _t2_to_native_plus_pe(B, L, E)(t2in, pe[:L])
    return out_T.transpose(2, 0, 1)


# speedup vs baseline: 1.7869x; 1.0855x over previous
import functools

import jax
import jax.numpy as jnp
from jax import lax
from jax.experimental import pallas as pl
from jax.experimental.pallas import tpu as pltpu
from jax.experimental.pallas import tpu_sc as plsc

NC, NS = 2, 16
CB = 2048
BB = 128


def _t1_table_to_rowmajor(V, E):
    nblk = (V + CB - 1) // CB
    CBH = CB // 2

    def body(x_ref, o_ref):
        eye = jnp.eye(E, dtype=jnp.float32)
        xt = jax.lax.dot_general(
            x_ref[...], eye, (((0,), (0,)), ((), ())),
            preferred_element_type=jnp.float32)
        o_ref[...] = jnp.concatenate([xt[:CBH], xt[CBH:]], axis=-1)

    return pl.pallas_call(
        body,
        grid=(nblk,),
        in_specs=[pl.BlockSpec((E, CB), lambda i: (0, i))],
        out_specs=pl.BlockSpec((CBH, 2 * E), lambda i: (i, 0)),
        out_shape=jax.ShapeDtypeStruct((nblk * CBH, 2 * E), jnp.float32),
    )


def _sc_gather(BL, V, E, per_w, chunk):
    n_chunks = per_w // chunk
    mesh = plsc.VectorSubcoreMesh(core_axis_name="c", subcore_axis_name="s")

    @functools.partial(
        pl.kernel,
        out_type=jax.ShapeDtypeStruct((BL, E), jnp.float32),
        mesh=mesh,
        scratch_types=[
            pltpu.VMEM((2, chunk), jnp.int32),
            pltpu.VMEM((2, chunk, E), jnp.float32),
            pltpu.SemaphoreType.DMA((2,)),
            pltpu.SemaphoreType.DMA((2,)),
            pltpu.SemaphoreType.DMA((2,)),
        ],
        compiler_params=pltpu.CompilerParams(use_tc_tiling_on_sc=False),
    )
    def k(seq_hbm, table_hbm, out_hbm, idx2, rows2, si, sg, ss):
        wid = lax.axis_index("s") * NC + lax.axis_index("c")
        w_base = wid * per_w

        def start_idx(g, b):
            pltpu.async_copy(
                seq_hbm.at[pl.ds(w_base + g * chunk, chunk)], idx2.at[b],
                si.at[b])

        def wait_idx(b):
            pltpu.make_async_copy(
                seq_hbm.at[pl.ds(0, chunk)], idx2.at[b], si.at[b]).wait()

        def start_gather(b):
            pltpu.async_copy(table_hbm.at[idx2.at[b]], rows2.at[b], sg.at[b])

        def wait_gather(b):
            pltpu.make_async_copy(
                table_hbm.at[idx2.at[b]], rows2.at[b], sg.at[b]).wait()

        def start_store(g, b):
            pltpu.async_copy(
                rows2.at[b], out_hbm.at[pl.ds(w_base + g * chunk, chunk)],
                ss.at[b])

        def wait_store(b):
            pltpu.make_async_copy(
                rows2.at[b], out_hbm.at[pl.ds(0, chunk)], ss.at[b]).wait()

        start_idx(0, 0)
        wait_idx(0)
        start_gather(0)
        start_idx(1, 1)

        def half(g, b):
            wait_idx(b)
            start_gather(b)
            wait_gather(1 - b)

            @pl.when(g + 1 < n_chunks)
            def _():
                start_idx(g + 1, 1 - b)

            start_store(g - 1, 1 - b)

        half(1, 1)

        def pair_body(p, carry):
            g = 2 * p
            wait_store(0)
            half(g, 0)
            wait_store(1)
            half(g + 1, 1)
            return carry

        lax.fori_loop(1, n_chunks // 2, pair_body, 0)

        b_last = (n_chunks - 1) % 2
        wait_gather(b_last)
        start_store(n_chunks - 1, b_last)
        wait_store(1 - b_last)
        wait_store(b_last)

    return k


def _t2_to_native_plus_pe(B, L, E):
    P = L // 2

    def body(x_ref, pe_ref, o_ref):
        x3 = x_ref[...].reshape(BB, P, 2 * E)
        pe_v = pe_ref[...]
        for p in range(P):
            xp = x3[:, p, :].T
            o_ref[2 * p] = xp[:E, :] + pe_v[2 * p][:, None]
            o_ref[2 * p + 1] = xp[E:, :] + pe_v[2 * p + 1][:, None]

    return pl.pallas_call(
        body,
        grid=(B // BB,),
        in_specs=[
            pl.BlockSpec((BB * P, 2 * E), lambda bi: (bi, 0)),
            pl.BlockSpec((L, E), lambda bi: (0, 0)),
        ],
        out_specs=pl.BlockSpec((L, E, BB), lambda bi: (0, 0, bi)),
        out_shape=jax.ShapeDtypeStruct((L, E, B), jnp.float32),
    )


def kernel(sequence, token_table, pe):
    B, L = sequence.shape
    V, E = token_table.shape
    BL = B * L
    n_workers = NC * NS
    per_w = BL // n_workers
    chunk = 800
    assert BL % n_workers == 0 and per_w % (2 * chunk) == 0
    assert B % BB == 0 and L % 2 == 0

    nblk = (V + CB - 1) // CB
    table_rm = _t1_table_to_rowmajor(V, E)(token_table.T)
    table_lin = table_rm.reshape(nblk * CB, E)

    t = sequence.astype(jnp.int32)
    c, j = t // CB, t % CB
    gidx = c * CB + 2 * (j % (CB // 2)) + j // (CB // 2)
    seq_flat = gidx.reshape(BL)
    flat = _sc_gather(BL, V, E, per_w, chunk)(seq_flat, table_lin)

    t2in = flat.reshape(B * L // 2, 2 * E)
    out_T = _t2_to_native_plus_pe(B, L, E)(t2in, pe[:L])
    return out_T.transpose(2, 0, 1)


# speedup vs baseline: 2.4133x; 1.3506x over previous
import functools

import jax
import jax.numpy as jnp
from jax import lax
from jax.experimental import pallas as pl
from jax.experimental.pallas import tpu as pltpu
from jax.experimental.pallas import tpu_sc as plsc

NC, NS = 2, 16
CB = 8192
BB = 128


def _t1_table_to_rowmajor(V, E):
    nblk = (V + CB - 1) // CB
    CBH = CB // 2

    def body(x_ref, o_ref):
        eye = jnp.eye(E, dtype=jnp.float32)
        xt = jax.lax.dot_general(
            x_ref[...], eye, (((0,), (0,)), ((), ())),
            preferred_element_type=jnp.float32)
        o_ref[...] = jnp.concatenate([xt[:CBH], xt[CBH:]], axis=-1)

    return pl.pallas_call(
        body,
        grid=(nblk,),
        in_specs=[pl.BlockSpec((E, CB), lambda i: (0, i))],
        out_specs=pl.BlockSpec((CBH, 2 * E), lambda i: (i, 0)),
        out_shape=jax.ShapeDtypeStruct((nblk * CBH, 2 * E), jnp.float32),
    )


def _sc_gather(BL, V, E, per_w, chunk):
    n_chunks = per_w // chunk
    mesh = plsc.VectorSubcoreMesh(core_axis_name="c", subcore_axis_name="s")

    @functools.partial(
        pl.kernel,
        out_type=jax.ShapeDtypeStruct((BL, E), jnp.float32),
        mesh=mesh,
        scratch_types=[
            pltpu.VMEM((2, chunk), jnp.int32),
            pltpu.VMEM((2, chunk, E), jnp.float32),
            pltpu.SemaphoreType.DMA((2,)),
            pltpu.SemaphoreType.DMA((2,)),
            pltpu.SemaphoreType.DMA((2,)),
        ],
        compiler_params=pltpu.CompilerParams(use_tc_tiling_on_sc=False),
    )
    def k(seq_hbm, table_hbm, out_hbm, idx2, rows2, si, sg, ss):
        wid = lax.axis_index("s") * NC + lax.axis_index("c")
        w_base = wid * per_w

        def start_idx(g, b):
            pltpu.async_copy(
                seq_hbm.at[pl.ds(w_base + g * chunk, chunk)], idx2.at[b],
                si.at[b])

        def wait_idx(b):
            pltpu.make_async_copy(
                seq_hbm.at[pl.ds(0, chunk)], idx2.at[b], si.at[b]).wait()

        def start_gather(b):
            pltpu.async_copy(table_hbm.at[idx2.at[b]], rows2.at[b], sg.at[b])

        def wait_gather(b):
            pltpu.make_async_copy(
                table_hbm.at[idx2.at[b]], rows2.at[b], sg.at[b]).wait()

        def start_store(g, b):
            pltpu.async_copy(
                rows2.at[b], out_hbm.at[pl.ds(w_base + g * chunk, chunk)],
                ss.at[b])

        def wait_store(b):
            pltpu.make_async_copy(
                rows2.at[b], out_hbm.at[pl.ds(0, chunk)], ss.at[b]).wait()

        start_idx(0, 0)
        wait_idx(0)
        start_gather(0)
        start_idx(1, 1)

        def half(g, b):
            wait_idx(b)
            start_gather(b)
            wait_gather(1 - b)

            @pl.when(g + 1 < n_chunks)
            def _():
                start_idx(g + 1, 1 - b)

            start_store(g - 1, 1 - b)

        half(1, 1)

        def pair_body(p, carry):
            g = 2 * p
            wait_store(0)
            half(g, 0)
            wait_store(1)
            half(g + 1, 1)
            return carry

        lax.fori_loop(1, n_chunks // 2, pair_body, 0)

        b_last = (n_chunks - 1) % 2
        wait_gather(b_last)
        start_store(n_chunks - 1, b_last)
        wait_store(1 - b_last)
        wait_store(b_last)

    return k


def _t2_to_native_plus_pe(B, L, E):
    P = L // 2

    def body(x_ref, pe_ref, o_ref):
        x3 = x_ref[...].reshape(BB, P, 2 * E)
        pe_v = pe_ref[...]
        for p in range(P):
            xp = x3[:, p, :].T
            o_ref[2 * p] = xp[:E, :] + pe_v[2 * p][:, None]
            o_ref[2 * p + 1] = xp[E:, :] + pe_v[2 * p + 1][:, None]

    return pl.pallas_call(
        body,
        grid=(B // BB,),
        in_specs=[
            pl.BlockSpec((BB * P, 2 * E), lambda bi: (bi, 0)),
            pl.BlockSpec((L, E), lambda bi: (0, 0)),
        ],
        out_specs=pl.BlockSpec((L, E, BB), lambda bi: (0, 0, bi)),
        out_shape=jax.ShapeDtypeStruct((L, E, B), jnp.float32),
    )


def kernel(sequence, token_table, pe):
    B, L = sequence.shape
    V, E = token_table.shape
    BL = B * L
    n_workers = NC * NS
    per_w = BL // n_workers
    chunk = 800
    assert BL % n_workers == 0 and per_w % (2 * chunk) == 0
    assert B % BB == 0 and L % 2 == 0

    nblk = (V + CB - 1) // CB
    table_rm = _t1_table_to_rowmajor(V, E)(token_table.T)
    table_lin = table_rm.reshape(nblk * CB, E)

    t = sequence.astype(jnp.int32)
    c, j = t // CB, t % CB
    gidx = c * CB + 2 * (j % (CB // 2)) + j // (CB // 2)
    seq_flat = gidx.reshape(BL)
    flat = _sc_gather(BL, V, E, per_w, chunk)(seq_flat, table_lin)

    t2in = flat.reshape(B * L // 2, 2 * E)
    out_T = _t2_to_native_plus_pe(B, L, E)(t2in, pe[:L])
    return out_T.transpose(2, 0, 1)


# speedup vs baseline: 2.6056x; 1.0796x over previous
import functools

import jax
import jax.numpy as jnp
from jax import lax
from jax.experimental import pallas as pl
from jax.experimental.pallas import tpu as pltpu
from jax.experimental.pallas import tpu_sc as plsc

NC, NS = 2, 16
CB = 20480
BB = 128


def _t1_table_to_rowmajor(V, E):
    nblk = (V + CB - 1) // CB
    CBH = CB // 2

    def body(x_ref, o_ref):
        eye = jnp.eye(E, dtype=jnp.float32)
        xt = jax.lax.dot_general(
            x_ref[...], eye, (((0,), (0,)), ((), ())),
            preferred_element_type=jnp.float32)
        o_ref[...] = jnp.concatenate([xt[:CBH], xt[CBH:]], axis=-1)

    return pl.pallas_call(
        body,
        grid=(nblk,),
        in_specs=[pl.BlockSpec((E, CB), lambda i: (0, i))],
        out_specs=pl.BlockSpec((CBH, 2 * E), lambda i: (i, 0)),
        out_shape=jax.ShapeDtypeStruct((nblk * CBH, 2 * E), jnp.float32),
    )


def _sc_gather(BL, V, E, per_w, chunk):
    n_chunks = per_w // chunk
    mesh = plsc.VectorSubcoreMesh(core_axis_name="c", subcore_axis_name="s")

    @functools.partial(
        pl.kernel,
        out_type=jax.ShapeDtypeStruct((BL, E), jnp.float32),
        mesh=mesh,
        scratch_types=[
            pltpu.VMEM((2, chunk), jnp.int32),
            pltpu.VMEM((2, chunk, E), jnp.float32),
            pltpu.SemaphoreType.DMA((2,)),
            pltpu.SemaphoreType.DMA((2,)),
            pltpu.SemaphoreType.DMA((2,)),
        ],
        compiler_params=pltpu.CompilerParams(use_tc_tiling_on_sc=False),
    )
    def k(seq_hbm, table_hbm, out_hbm, idx2, rows2, si, sg, ss):
        wid = lax.axis_index("s") * NC + lax.axis_index("c")
        w_base = wid * per_w

        def start_idx(g, b):
            pltpu.async_copy(
                seq_hbm.at[pl.ds(w_base + g * chunk, chunk)], idx2.at[b],
                si.at[b])

        def wait_idx(b):
            pltpu.make_async_copy(
                seq_hbm.at[pl.ds(0, chunk)], idx2.at[b], si.at[b]).wait()

        def start_gather(b):
            pltpu.async_copy(table_hbm.at[idx2.at[b]], rows2.at[b], sg.at[b])

        def wait_gather(b):
            pltpu.make_async_copy(
                table_hbm.at[idx2.at[b]], rows2.at[b], sg.at[b]).wait()

        def start_store(g, b):
            pltpu.async_copy(
                rows2.at[b], out_hbm.at[pl.ds(w_base + g * chunk, chunk)],
                ss.at[b])

        def wait_store(b):
            pltpu.make_async_copy(
                rows2.at[b], out_hbm.at[pl.ds(0, chunk)], ss.at[b]).wait()

        start_idx(0, 0)
        wait_idx(0)
        start_gather(0)
        start_idx(1, 1)

        def half(g, b):
            wait_idx(b)
            start_gather(b)
            wait_gather(1 - b)

            @pl.when(g + 1 < n_chunks)
            def _():
                start_idx(g + 1, 1 - b)

            start_store(g - 1, 1 - b)

        half(1, 1)

        def pair_body(p, carry):
            g = 2 * p
            wait_store(0)
            half(g, 0)
            wait_store(1)
            half(g + 1, 1)
            return carry

        lax.fori_loop(1, n_chunks // 2, pair_body, 0)

        b_last = (n_chunks - 1) % 2
        wait_gather(b_last)
        start_store(n_chunks - 1, b_last)
        wait_store(1 - b_last)
        wait_store(b_last)

    return k


def _t2_to_native_plus_pe(B, L, E):
    P = L // 2

    def body(x_ref, pe_ref, o_ref):
        x3 = x_ref[...].reshape(BB, P, 2 * E)
        pe_v = pe_ref[...]
        for p in range(P):
            xp = x3[:, p, :].T
            o_ref[2 * p] = xp[:E, :] + pe_v[2 * p][:, None]
            o_ref[2 * p + 1] = xp[E:, :] + pe_v[2 * p + 1][:, None]

    return pl.pallas_call(
        body,
        grid=(B // BB,),
        in_specs=[
            pl.BlockSpec((BB * P, 2 * E), lambda bi: (bi, 0)),
            pl.BlockSpec((L, E), lambda bi: (0, 0)),
        ],
        out_specs=pl.BlockSpec((L, E, BB), lambda bi: (0, 0, bi)),
        out_shape=jax.ShapeDtypeStruct((L, E, B), jnp.float32),
    )


def kernel(sequence, token_table, pe):
    B, L = sequence.shape
    V, E = token_table.shape
    BL = B * L
    n_workers = NC * NS
    per_w = BL // n_workers
    chunk = 800
    assert BL % n_workers == 0 and per_w % (2 * chunk) == 0
    assert B % BB == 0 and L % 2 == 0

    nblk = (V + CB - 1) // CB
    table_rm = _t1_table_to_rowmajor(V, E)(token_table.T)
    table_lin = table_rm.reshape(nblk * CB, E)

    t = sequence.astype(jnp.int32)
    c, j = t // CB, t % CB
    gidx = c * CB + 2 * (j % (CB // 2)) + j // (CB // 2)
    seq_flat = gidx.reshape(BL)
    flat = _sc_gather(BL, V, E, per_w, chunk)(seq_flat, table_lin)

    t2in = flat.reshape(B * L // 2, 2 * E)
    out_T = _t2_to_native_plus_pe(B, L, E)(t2in, pe[:L])
    return out_T.transpose(2, 0, 1)
